# baseline (speedup 1.0000x reference)
import functools

import jax
import jax.numpy as jnp
from jax import lax
from jax.experimental import pallas as pl
from jax.experimental.pallas import tpu as pltpu
from jax.experimental.pallas import tpu_sc as plsc

N_NODES = 10000
NPAD = 10240
N_EDGES = 320000
NB = 80
EB = 128
EP = 32 * NB * EB
TRASH = 10000
ROWS_PER_SUB = NPAD // 16
NBUF = 4

_mesh = plsc.VectorSubcoreMesh(core_axis_name="c", subcore_axis_name="s")


@functools.partial(
    pl.kernel,
    out_type=jax.ShapeDtypeStruct((2, NPAD, 128), jnp.float32),
    mesh=_mesh,
    scratch_types=[
        pltpu.VMEM((NB, EB), jnp.int32),
        pltpu.VMEM_SHARED((NPAD, 128), jnp.float32),
    ] + [pltpu.VMEM((EB, 128), jnp.float32) for _ in range(2)]
      + [pltpu.VMEM((EB,), jnp.int32) for _ in range(4)]
      + [pltpu.SemaphoreType.DMA for _ in range(8)],
)
def _sc_aggregate(table_hbm, src_hbm, dst_hbm, zeros_hbm, out_hbm,
                  src_v, acc, *rest):
    bufs = rest[0:2]
    dsts = rest[2:6]
    gsem = rest[6:8]
    ssem = rest[8:10]
    isem = rest[10:14]
    c = lax.axis_index("c")
    s = lax.axis_index("s")
    wid = s * 2 + c
    pltpu.sync_copy(src_hbm.at[wid], src_v)
    pltpu.sync_copy(zeros_hbm, acc.at[pl.ds(s * ROWS_PER_SUB, ROWS_PER_SUB)])
    plsc.subcore_barrier()

    def start_idx(b, k):
        pltpu.async_copy(dst_hbm.at[wid].at[b], dsts[k], isem[k])

    def wait_idx(k):
        pltpu.make_async_copy(dst_hbm.at[0].at[0], dsts[k], isem[k]).wait()

    def start_gather(b, j):
        pltpu.async_copy(table_hbm.at[src_v.at[b]], bufs[j], gsem[j])

    def wait_gather(j):
        pltpu.make_async_copy(table_hbm.at[pl.ds(0, EB)], bufs[j], gsem[j]).wait()

    def start_scatter(k, j):
        pltpu.async_copy(bufs[j], acc.at[dsts[k]], ssem[j], add=True)

    def wait_scatter(j):
        pltpu.make_async_copy(bufs[j], acc.at[pl.ds(0, EB)], ssem[j]).wait()

    for k in range(4):
        start_idx(k, k)
    for j in range(2):
        start_gather(j, j)

    def outer(i, carry):
        for k in range(4):
            b = i * 4 + k
            j = k % 2
            wait_gather(j)
            wait_idx(k)
            start_scatter(k, j)
            wait_scatter(j)
            start_idx(b + 4, k)
            start_gather(b + 2, j)
        return carry

    lax.fori_loop(0, NB // 4 - 1, outer, 0)
    for k in range(4):
        b = NB - 4 + k
        j = k % 2
        wait_gather(j)
        wait_idx(k)
        start_scatter(k, j)
        wait_scatter(j)
        if b + 2 < NB:
            start_gather(b + 2, j)

    plsc.subcore_barrier()
    pltpu.sync_copy(
        acc.at[pl.ds(s * ROWS_PER_SUB, ROWS_PER_SUB)],
        out_hbm.at[c].at[pl.ds(s * ROWS_PER_SUB, ROWS_PER_SUB)],
    )


def _mm_body(x_ref, w_ref, o_ref):
    o_ref[...] = jax.lax.dot_general(
        x_ref[...], w_ref[...], (((1,), (0,)), ((), ())),
        preferred_element_type=jnp.float32,
        precision=jax.lax.Precision.HIGHEST,
    )


def _matmul(x, w):
    m, k = x.shape
    _, n = w.shape
    bm = 2000
    n_pad = ((n + 127) // 128) * 128
    if n_pad != n:
        w = jnp.pad(w, ((0, 0), (0, n_pad - n)))
    return pl.pallas_call(
        _mm_body,
        grid=(m // bm,),
        in_specs=[
            pl.BlockSpec((bm, k), lambda i: (i, 0)),
            pl.BlockSpec((k, n_pad), lambda i: (0, 0)),
        ],
        out_specs=pl.BlockSpec((bm, n_pad), lambda i: (i, 0)),
        out_shape=jax.ShapeDtypeStruct((m, n_pad), jnp.float32),
    )(x, w)


def kernel(x, edge_index, W1, b1, W2, b2, W3, b3, W4, b4, W5, b5, W6, b6):
    src = edge_index[0].astype(jnp.int32)
    dst = edge_index[1].astype(jnp.int32)
    pad = EP - N_EDGES
    src_p = jnp.concatenate([src, jnp.zeros((pad,), jnp.int32)]).reshape(32, NB, EB)
    dst_p = jnp.concatenate([dst, jnp.full((pad,), TRASH, jnp.int32)]).reshape(32, NB, EB)

    zeros128 = jnp.zeros((ROWS_PER_SUB, 128), jnp.float32)

    ones_tab = jnp.ones((N_NODES, 128), jnp.float32)
    dpart = _sc_aggregate(ones_tab, src_p, dst_p, zeros128)
    deg = dpart[0, :N_NODES, 0] + dpart[1, :N_NODES, 0] + 1.0
    dinv = lax.rsqrt(jnp.maximum(deg, 1e-12))[:, None]

    def aggregate(hs):
        f = hs.shape[1]
        outs = []
        for ci in range(f // 128):
            part = _sc_aggregate(hs[:, ci * 128:(ci + 1) * 128], src_p, dst_p, zeros128)
            outs.append(part[0, :N_NODES] + part[1, :N_NODES])
        return outs[0] if len(outs) == 1 else jnp.concatenate(outs, axis=1)

    xs = dinv * x
    u = dinv * (aggregate(xs) + xs)
    h = jax.nn.relu(_matmul(u, W1)[:, :640] + b1)

    for W, b, act in ((W2, b2, True), (W3, b3, True), (W4, b4, True),
                      (W5, b5, True), (W6, b6, False)):
        n_out = W.shape[1]
        t = _matmul(h, W)
        hs = dinv * t
        h = dinv * (aggregate(hs) + hs)[:, :n_out] + b
        if act:
            h = jax.nn.relu(h)
    return jax.nn.log_softmax(h, axis=1)
